# baseline (speedup 1.0000x reference)
import functools

import jax
import jax.numpy as jnp
from jax import lax
from jax.experimental import pallas as pl
from jax.experimental.pallas import tpu as pltpu
from jax.experimental.pallas import tpu_sc as plsc

NC = 2
NS = 16
NW = NC * NS
LANES = 16
CH = 96

_mesh = plsc.VectorSubcoreMesh(core_axis_name="c", subcore_axis_name="s")


def _build_deg_kernel(np_, ept):
    nps = np_ // NS
    nchunk = ept // CH

    @functools.partial(
        pl.kernel,
        out_type=jax.ShapeDtypeStruct((NC * np_,), jnp.float32),
        mesh=_mesh,
        compiler_params=pltpu.CompilerParams(needs_layout_passes=False),
        scratch_types=[
            pltpu.VMEM_SHARED((np_,), jnp.float32),
            pltpu.VMEM((ept,), jnp.int32),
            pltpu.VMEM((ept,), jnp.float32),
            pltpu.VMEM((CH,), jnp.int32),
            pltpu.VMEM((CH,), jnp.float32),
            pltpu.VMEM((nps,), jnp.float32),
        ],
    )
    def deg_kernel(dst_hbm, w_hbm, out_hbm, acc, dstv, wv, dstc, wc, zbuf):
        c = lax.axis_index("c")
        s = lax.axis_index("s")
        wid = s * NC + c
        zero = jnp.zeros((LANES,), jnp.float32)

        @pl.loop(0, nps // LANES)
        def _(i):
            zbuf[pl.ds(i * LANES, LANES)] = zero

        pltpu.sync_copy(zbuf, acc.at[pl.ds(s * nps, nps)])
        pltpu.sync_copy(dst_hbm.at[pl.ds(wid * ept, ept)], dstv)
        pltpu.sync_copy(w_hbm.at[pl.ds(wid * ept, ept)], wv)
        plsc.subcore_barrier()

        @pl.loop(0, nchunk)
        def _(k):
            for g in range(CH // LANES):
                dstc[pl.ds(g * LANES, LANES)] = (
                    dstv[pl.ds(k * CH + g * LANES, LANES)])
                wc[pl.ds(g * LANES, LANES)] = (
                    wv[pl.ds(k * CH + g * LANES, LANES)])
            pltpu.sync_copy(wc, acc.at[dstc], add=True)

        plsc.subcore_barrier()
        pltpu.sync_copy(acc.at[pl.ds(s * nps, nps)],
                        out_hbm.at[pl.ds(c * np_ + s * nps, nps)])

    return deg_kernel


def _build_spmm_kernel(np_, d, ept, shift):
    nps = np_ // NS
    nchunk = ept // CH
    mask = (1 << shift) - 1

    zr = 64

    @functools.partial(
        pl.kernel,
        out_type=jax.ShapeDtypeStruct((NC, np_, d), jnp.float32),
        mesh=_mesh,
        compiler_params=pltpu.CompilerParams(needs_layout_passes=False,
                                             use_tc_tiling_on_sc=False),
        scratch_types=[
            pltpu.VMEM_SHARED((np_, d), jnp.float32),
            pltpu.VMEM((np_,), jnp.float32),
            pltpu.VMEM((ept,), jnp.int32),
            pltpu.VMEM((4, CH // 2), jnp.int32),
            pltpu.VMEM((2, CH), jnp.int32),
            pltpu.VMEM((2, CH), jnp.float32),
            pltpu.VMEM((2, CH), jnp.float32),
            pltpu.VMEM((CH, d), jnp.float32),
            pltpu.VMEM((CH, d), jnp.float32),
            pltpu.SemaphoreType.DMA,
            pltpu.SemaphoreType.DMA,
            pltpu.SemaphoreType.DMA,
            pltpu.SemaphoreType.DMA,
            pltpu.SemaphoreType.DMA,
            pltpu.SemaphoreType.DMA,
        ],
    )
    def spmm_kernel(pk_hbm, w_hbm, dis_hbm, h_hbm, out_hbm,
                    acc, disv, pkv, srcc, dstc, wc, nvb,
                    rows_a, rows_b, gsem_a, gsem_b, g2sem_a, g2sem_b,
                    asem_a, asem_b):
        c = lax.axis_index("c")
        s = lax.axis_index("s")
        wid = s * NC + c
        zero = jnp.zeros((LANES,), jnp.float32)
        rows = (rows_a, rows_b)
        gsem = (gsem_a, gsem_b)
        g2sem = (g2sem_a, g2sem_b)
        asem = (asem_a, asem_b)

        @pl.loop(0, zr)
        def _(i):
            for j in range(d // LANES):
                rows_a[i, pl.ds(j * LANES, LANES)] = zero

        @pl.loop(0, nps // zr)
        def _(i):
            pltpu.sync_copy(rows_a.at[pl.ds(0, zr)],
                            acc.at[pl.ds(s * nps + i * zr, zr)])

        pltpu.sync_copy(dis_hbm, disv)
        pltpu.sync_copy(pk_hbm.at[pl.ds(wid * ept, ept)], pkv)
        plsc.subcore_barrier()

        def start_fetch(k, p):
            hc = CH // 2

            @pl.loop(0, CH // LANES)
            def _(g):
                p16 = pkv[pl.ds(k * CH + g * LANES, LANES)]
                half = 2 * p + g // (hc // LANES)
                off = (g % (hc // LANES)) * LANES
                srcc[half, pl.ds(off, LANES)] = p16 & mask
                dstc[p, pl.ds(g * LANES, LANES)] = (
                    lax.shift_right_logical(p16, shift))

            pltpu.async_copy(h_hbm.at[srcc.at[2 * p]],
                             rows[p].at[pl.ds(0, hc)], gsem[p])
            pltpu.async_copy(h_hbm.at[srcc.at[2 * p + 1]],
                             rows[p].at[pl.ds(hc, hc)], g2sem[p])
            pltpu.async_copy(w_hbm.at[pl.ds(wid * ept + k * CH, CH)],
                             wc.at[p], asem[p])

        def process(k, p):
            pltpu.make_async_copy(
                w_hbm.at[pl.ds(wid * ept + k * CH, CH)],
                wc.at[p], asem[p]).wait()

            hc = CH // 2

            @pl.loop(0, CH // LANES)
            def _(g):
                half = 2 * p + g // (hc // LANES)
                off = (g % (hc // LANES)) * LANES
                s16 = srcc[half, pl.ds(off, LANES)]
                d16 = dstc[p, pl.ds(g * LANES, LANES)]
                w16 = wc[p, pl.ds(g * LANES, LANES)]
                nv = plsc.load_gather(disv, [s16]) * w16
                nvb[p, pl.ds(g * LANES, LANES)] = (
                    nv * plsc.load_gather(disv, [d16]))

            pltpu.make_async_copy(h_hbm.at[srcc.at[2 * p]],
                                  rows[p].at[pl.ds(0, hc)], gsem[p]).wait()
            pltpu.make_async_copy(h_hbm.at[srcc.at[2 * p + 1]],
                                  rows[p].at[pl.ds(hc, hc)],
                                  g2sem[p]).wait()

            pidx = jnp.full((LANES,), p, jnp.int32)

            @pl.loop(0, CH)
            def _(i):
                nb = plsc.load_gather(
                    nvb, [pidx, jnp.zeros((LANES,), jnp.int32) + i])
                for j in range(d // LANES):
                    rows[p][i, pl.ds(j * LANES, LANES)] = (
                        rows[p][i, pl.ds(j * LANES, LANES)] * nb)

            pltpu.sync_copy(rows[p], acc.at[dstc.at[p]], add=True)

        start_fetch(0, 0)

        @pl.loop(0, nchunk - 1, step=2)
        def _(t):
            start_fetch(t + 1, 1)
            process(t, 0)
            start_fetch(t + 2, 0)
            process(t + 1, 1)

        process(nchunk - 1, 0)
        plsc.subcore_barrier()

        pltpu.sync_copy(acc.at[pl.ds(s * nps, nps)],
                        out_hbm.at[c, pl.ds(s * nps, nps)])

    return spmm_kernel


def _dis_tc_kernel(deg_ref, o_ref):
    deg = deg_ref[0:1, :] + deg_ref[1:2, :]
    o_ref[...] = jnp.where(deg > 0,
                           lax.rsqrt(jnp.maximum(deg, 1e-12)),
                           0.0)


def _mm_tc_kernel(x_ref, w_ref, o_ref):
    o_ref[...] = lax.dot_general(
        x_ref[...], w_ref[...], (((1,), (1,)), ((), ())),
        preferred_element_type=jnp.float32)


def _final_tc_kernel(s_ref, w2_ref, b1_ref, b2_ref, o_ref):
    a = s_ref[0] + s_ref[1]
    acc = lax.dot_general(a, w2_ref[...], (((1,), (1,)), ((), ())),
                          preferred_element_type=jnp.float32)
    bias = lax.dot_general(b1_ref[...], w2_ref[...], (((1,), (1,)), ((), ())),
                           preferred_element_type=jnp.float32)
    o_ref[...] = acc + bias + b2_ref[...]


def kernel(x, edge_index, edge_weight, W1, b1, W2, b2):
    n, d = x.shape
    e = edge_index.shape[1]

    np_ = ((n + 1023) // 1024) * 1024
    nchunk = -(-(e + n) // (NW * CH))
    if nchunk % 2 == 0:
        nchunk += 1
    ept = nchunk * CH
    ep = ept * NW
    pad = ep - e - n

    shift = (n - 1).bit_length()
    src = edge_index[0].astype(jnp.int32)
    dst = edge_index[1].astype(jnp.int32)
    loop_idx = jnp.arange(n, dtype=jnp.int32)
    pad_idx = jnp.arange(pad, dtype=jnp.int32) % jnp.int32(n)
    src_all = jnp.concatenate([src, loop_idx, pad_idx])
    dst_all = jnp.concatenate([dst, loop_idx, pad_idx])
    pk_all = src_all | (dst_all << shift)
    w_all = jnp.concatenate([
        edge_weight.astype(jnp.float32),
        jnp.ones((n,), jnp.float32),
        jnp.zeros((pad,), jnp.float32),
    ])
    x_pad = jnp.pad(x, ((0, np_ - n), (0, 0)))

    deg_parts = _build_deg_kernel(np_, ept)(dst_all, w_all)

    dis = pl.pallas_call(
        _dis_tc_kernel,
        out_shape=jax.ShapeDtypeStruct((1, np_), jnp.float32),
    )(deg_parts.reshape(NC, np_)).reshape(np_)

    bm = 1024
    h = pl.pallas_call(
        _mm_tc_kernel,
        grid=(np_ // bm,),
        in_specs=[pl.BlockSpec((bm, d), lambda i: (i, 0)),
                  pl.BlockSpec((d, d), lambda i: (0, 0))],
        out_specs=pl.BlockSpec((bm, d), lambda i: (i, 0)),
        out_shape=jax.ShapeDtypeStruct((np_, d), jnp.float32),
    )(x_pad, W1)

    s_parts = _build_spmm_kernel(np_, d, ept, shift)(pk_all, w_all, dis, h)

    out_full = pl.pallas_call(
        _final_tc_kernel,
        grid=(np_ // bm,),
        in_specs=[pl.BlockSpec((NC, bm, d), lambda i: (0, i, 0)),
                  pl.BlockSpec((d, d), lambda i: (0, 0)),
                  pl.BlockSpec((1, d), lambda i: (0, 0)),
                  pl.BlockSpec((1, d), lambda i: (0, 0))],
        out_specs=pl.BlockSpec((bm, d), lambda i: (i, 0)),
        out_shape=jax.ShapeDtypeStruct((np_, d), jnp.float32),
    )(s_parts, W2, b1.reshape(1, d), b2.reshape(1, d))

    return out_full[:n]


# speedup vs baseline: 1.0186x; 1.0186x over previous
import functools

import jax
import jax.numpy as jnp
from jax import lax
from jax.experimental import pallas as pl
from jax.experimental.pallas import tpu as pltpu
from jax.experimental.pallas import tpu_sc as plsc

NC = 2
NS = 16
NW = NC * NS
LANES = 16
CH = 96

_mesh = plsc.VectorSubcoreMesh(core_axis_name="c", subcore_axis_name="s")


def _build_deg_kernel(np_, ept):
    nps = np_ // NS
    nchunk = ept // CH

    @functools.partial(
        pl.kernel,
        out_type=jax.ShapeDtypeStruct((NC * np_,), jnp.float32),
        mesh=_mesh,
        compiler_params=pltpu.CompilerParams(needs_layout_passes=False),
        scratch_types=[
            pltpu.VMEM_SHARED((np_,), jnp.float32),
            pltpu.VMEM((ept,), jnp.int32),
            pltpu.VMEM((ept,), jnp.float32),
            pltpu.VMEM((CH,), jnp.int32),
            pltpu.VMEM((CH,), jnp.float32),
            pltpu.VMEM((nps,), jnp.float32),
        ],
    )
    def deg_kernel(dst_hbm, w_hbm, out_hbm, acc, dstv, wv, dstc, wc, zbuf):
        c = lax.axis_index("c")
        s = lax.axis_index("s")
        wid = s * NC + c
        zero = jnp.zeros((LANES,), jnp.float32)

        @pl.loop(0, nps // LANES)
        def _(i):
            zbuf[pl.ds(i * LANES, LANES)] = zero

        pltpu.sync_copy(zbuf, acc.at[pl.ds(s * nps, nps)])
        pltpu.sync_copy(dst_hbm.at[pl.ds(wid * ept, ept)], dstv)
        pltpu.sync_copy(w_hbm.at[pl.ds(wid * ept, ept)], wv)
        plsc.subcore_barrier()

        @pl.loop(0, nchunk)
        def _(k):
            for g in range(CH // LANES):
                dstc[pl.ds(g * LANES, LANES)] = (
                    dstv[pl.ds(k * CH + g * LANES, LANES)])
                wc[pl.ds(g * LANES, LANES)] = (
                    wv[pl.ds(k * CH + g * LANES, LANES)])
            pltpu.sync_copy(wc, acc.at[dstc], add=True)

        plsc.subcore_barrier()
        pltpu.sync_copy(acc.at[pl.ds(s * nps, nps)],
                        out_hbm.at[pl.ds(c * np_ + s * nps, nps)])

    return deg_kernel


def _build_spmm_kernel(np_, d, ept, shift):
    nps = np_ // NS
    nchunk = ept // CH
    mask = (1 << shift) - 1

    zr = 64

    @functools.partial(
        pl.kernel,
        out_type=jax.ShapeDtypeStruct((NC, np_, d), jnp.float32),
        mesh=_mesh,
        compiler_params=pltpu.CompilerParams(needs_layout_passes=False),
        scratch_types=[
            pltpu.VMEM_SHARED((np_, d), jnp.float32),
            pltpu.VMEM((np_,), jnp.float32),
            pltpu.VMEM((ept,), jnp.int32),
            pltpu.VMEM((4, CH // 2), jnp.int32),
            pltpu.VMEM((2, CH), jnp.int32),
            pltpu.VMEM((2, CH), jnp.float32),
            pltpu.VMEM((2, CH), jnp.float32),
            pltpu.VMEM((CH, d), jnp.float32),
            pltpu.VMEM((CH, d), jnp.float32),
            pltpu.SemaphoreType.DMA,
            pltpu.SemaphoreType.DMA,
            pltpu.SemaphoreType.DMA,
            pltpu.SemaphoreType.DMA,
            pltpu.SemaphoreType.DMA,
            pltpu.SemaphoreType.DMA,
        ],
    )
    def spmm_kernel(pk_hbm, w_hbm, dis_hbm, h_hbm, out_hbm,
                    acc, disv, pkv, srcc, dstc, wc, nvb,
                    rows_a, rows_b, gsem_a, gsem_b, g2sem_a, g2sem_b,
                    asem_a, asem_b):
        c = lax.axis_index("c")
        s = lax.axis_index("s")
        wid = s * NC + c
        zero = jnp.zeros((LANES,), jnp.float32)
        rows = (rows_a, rows_b)
        gsem = (gsem_a, gsem_b)
        g2sem = (g2sem_a, g2sem_b)
        asem = (asem_a, asem_b)

        @pl.loop(0, zr)
        def _(i):
            for j in range(d // LANES):
                rows_a[i, pl.ds(j * LANES, LANES)] = zero

        @pl.loop(0, nps // zr)
        def _(i):
            pltpu.sync_copy(rows_a.at[pl.ds(0, zr)],
                            acc.at[pl.ds(s * nps + i * zr, zr)])

        pltpu.sync_copy(dis_hbm, disv)
        pltpu.sync_copy(pk_hbm.at[pl.ds(wid * ept, ept)], pkv)
        plsc.subcore_barrier()

        def start_fetch(k, p):
            hc = CH // 2

            @pl.loop(0, CH // LANES)
            def _(g):
                p16 = pkv[pl.ds(k * CH + g * LANES, LANES)]
                half = 2 * p + g // (hc // LANES)
                off = (g % (hc // LANES)) * LANES
                srcc[half, pl.ds(off, LANES)] = p16 & mask
                dstc[p, pl.ds(g * LANES, LANES)] = (
                    lax.shift_right_logical(p16, shift))

            pltpu.async_copy(h_hbm.at[srcc.at[2 * p]],
                             rows[p].at[pl.ds(0, hc)], gsem[p])
            pltpu.async_copy(h_hbm.at[srcc.at[2 * p + 1]],
                             rows[p].at[pl.ds(hc, hc)], g2sem[p])
            pltpu.async_copy(w_hbm.at[pl.ds(wid * ept + k * CH, CH)],
                             wc.at[p], asem[p])

        def process(k, p):
            pltpu.make_async_copy(
                w_hbm.at[pl.ds(wid * ept + k * CH, CH)],
                wc.at[p], asem[p]).wait()

            hc = CH // 2

            @pl.loop(0, CH // LANES)
            def _(g):
                half = 2 * p + g // (hc // LANES)
                off = (g % (hc // LANES)) * LANES
                s16 = srcc[half, pl.ds(off, LANES)]
                d16 = dstc[p, pl.ds(g * LANES, LANES)]
                w16 = wc[p, pl.ds(g * LANES, LANES)]
                nv = plsc.load_gather(disv, [s16]) * w16
                nvb[p, pl.ds(g * LANES, LANES)] = (
                    nv * plsc.load_gather(disv, [d16]))

            pltpu.make_async_copy(h_hbm.at[srcc.at[2 * p]],
                                  rows[p].at[pl.ds(0, hc)], gsem[p]).wait()
            pltpu.make_async_copy(h_hbm.at[srcc.at[2 * p + 1]],
                                  rows[p].at[pl.ds(hc, hc)],
                                  g2sem[p]).wait()

            pidx = jnp.full((LANES,), p, jnp.int32)

            @pl.loop(0, CH)
            def _(i):
                nb = plsc.load_gather(
                    nvb, [pidx, jnp.zeros((LANES,), jnp.int32) + i])
                for j in range(d // LANES):
                    rows[p][i, pl.ds(j * LANES, LANES)] = (
                        rows[p][i, pl.ds(j * LANES, LANES)] * nb)

            pltpu.sync_copy(rows[p], acc.at[dstc.at[p]], add=True)

        start_fetch(0, 0)

        @pl.loop(0, nchunk - 1, step=2)
        def _(t):
            start_fetch(t + 1, 1)
            process(t, 0)
            start_fetch(t + 2, 0)
            process(t + 1, 1)

        process(nchunk - 1, 0)
        plsc.subcore_barrier()

        pltpu.sync_copy(acc.at[pl.ds(s * nps, nps)],
                        out_hbm.at[c, pl.ds(s * nps, nps)])

    return spmm_kernel


def _mm_dis_tc_kernel(x_ref, w_ref, deg_ref, h_ref, dis_ref):
    h_ref[...] = lax.dot_general(
        x_ref[...], w_ref[...], (((1,), (1,)), ((), ())),
        preferred_element_type=jnp.float32)
    deg = deg_ref[0:1, :] + deg_ref[1:2, :]
    dis_ref[...] = jnp.where(deg > 0,
                             lax.rsqrt(jnp.maximum(deg, 1e-12)),
                             0.0)


def _final_tc_kernel(s_ref, w2_ref, b1_ref, b2_ref, o_ref):
    nn = o_ref.shape[0]
    a = s_ref[0, 0:nn, :] + s_ref[1, 0:nn, :]
    acc = lax.dot_general(a, w2_ref[...], (((1,), (1,)), ((), ())),
                          preferred_element_type=jnp.float32)
    bias = lax.dot_general(b1_ref[...], w2_ref[...], (((1,), (1,)), ((), ())),
                           preferred_element_type=jnp.float32)
    o_ref[...] = acc + bias + b2_ref[...]


def kernel(x, edge_index, edge_weight, W1, b1, W2, b2):
    n, d = x.shape
    e = edge_index.shape[1]

    np_ = ((n + 1023) // 1024) * 1024
    nchunk = -(-(e + n) // (NW * CH))
    if nchunk % 2 == 0:
        nchunk += 1
    ept = nchunk * CH
    ep = ept * NW
    pad = ep - e - n

    shift = (n - 1).bit_length()
    src = edge_index[0].astype(jnp.int32)
    dst = edge_index[1].astype(jnp.int32)
    loop_idx = jnp.arange(n, dtype=jnp.int32)
    pad_idx = jnp.arange(pad, dtype=jnp.int32) % jnp.int32(n)
    src_all = jnp.concatenate([src, loop_idx, pad_idx])
    dst_all = jnp.concatenate([dst, loop_idx, pad_idx])
    pk_all = src_all | (dst_all << shift)
    w_all = jnp.concatenate([
        edge_weight.astype(jnp.float32),
        jnp.ones((n,), jnp.float32),
        jnp.zeros((pad,), jnp.float32),
    ])
    deg_parts = _build_deg_kernel(np_, ept)(dst_all, w_all)

    h, dis2 = pl.pallas_call(
        _mm_dis_tc_kernel,
        out_shape=(jax.ShapeDtypeStruct((n, d), jnp.float32),
                   jax.ShapeDtypeStruct((1, np_), jnp.float32)),
    )(x, W1, deg_parts.reshape(NC, np_))

    s_parts = _build_spmm_kernel(np_, d, ept, shift)(
        pk_all, w_all, dis2.reshape(np_), h)

    out = pl.pallas_call(
        _final_tc_kernel,
        out_shape=jax.ShapeDtypeStruct((n, d), jnp.float32),
    )(s_parts, W2, b1.reshape(1, d), b2.reshape(1, d))

    return out
